# Initial kernel scaffold; baseline (speedup 1.0000x reference)
import jax
import jax.numpy as jnp
from jax.experimental import pallas as pl


def kernel(input, x, edge_index, W_lin, b_lin, gamma, beta, W1, b1, W2, b2, W3, b3, Wm, bm):
    raise NotImplementedError("write your pallas kernel here")


# speedup vs baseline: 3.3421x; 3.3421x over previous
import functools

import jax
import jax.numpy as jnp
from jax import lax
from jax.experimental import pallas as pl
from jax.experimental.pallas import tpu as pltpu
from jax.experimental.pallas import tpu_sc as plsc

F32 = jnp.float32
NC = 2
NS = 16
NW = NC * NS
CH = 128

_HIGH = jax.lax.Precision.HIGHEST


def _dot(a, b):
    return jnp.dot(a, b, preferred_element_type=F32, precision=_HIGH)


def _node_feat_body(inp_ref, wl_ref, bl_ref, g_ref, b_ref, out_ref):
    h = _dot(inp_ref[...], wl_ref[...]) + bl_ref[...]
    s = jnp.maximum(h, 0.0)
    mu = jnp.mean(s, axis=1, keepdims=True)
    var = jnp.mean((s - mu) ** 2, axis=1, keepdims=True)
    out_ref[...] = (s - mu) / jnp.sqrt(var + 1e-3) * g_ref[...] + b_ref[...]


def _edge_update_body(fi_ref, fj_ref, ci_ref, cj_ref, w1a_ref, w1b_ref,
                      wm_ref, w2r0_ref, w2r1_ref, w3r0_ref, w3r1_ref,
                      bsum_ref, bm_ref, out_ref):
    fi = fi_ref[...]
    fj = fj_ref[...]
    ci = ci_ref[...]
    cj = cj_ref[...]
    gi = 1.0 / (jnp.sqrt(jnp.sum(fi * fi, axis=1, keepdims=True)) + 1e-4)
    gj = 1.0 / (jnp.sqrt(jnp.sum(fj * fj, axis=1, keepdims=True)) + 1e-4)
    zf = jnp.clip(jnp.sum(fi * fj, axis=1, keepdims=True) * gi * gj,
                  -0.99, 0.99)
    sf = jnp.sqrt(1.0 - zf * zf)
    hi = 1.0 / (jnp.sqrt(jnp.sum(ci * ci, axis=1, keepdims=True)) + 1e-4)
    hj = 1.0 / (jnp.sqrt(jnp.sum(cj * cj, axis=1, keepdims=True)) + 1e-4)
    zx = jnp.clip(jnp.sum(ci * cj, axis=1, keepdims=True) * hi * hj,
                  -0.99, 0.99)
    sx = jnp.sqrt(1.0 - zx * zx)
    m = (_dot(fi, w1a_ref[...]) + _dot(fj, w1b_ref[...]) + bsum_ref[...]
         + zf * w2r0_ref[...] + sf * w2r1_ref[...]
         + zx * w3r0_ref[...] + sx * w3r1_ref[...])
    msg = jnp.maximum(m, 0.0)
    u = _dot(msg, wm_ref[...]) + bm_ref[...]
    out_ref[...] = (ci - cj) * u


def _normalize_body(x_ref, p_ref, out_ref):
    coord = x_ref[...] + p_ref[0] + p_ref[1]
    nrm = jnp.sqrt(jnp.sum(coord * coord, axis=1, keepdims=True))
    out_ref[...] = coord / jnp.maximum(nrm, 1e-6)


def _acos(z):
    ax = jnp.abs(z)
    p = jnp.float32(-0.0012624911)
    for c in (0.0066700901, -0.0170881256, 0.0308918810, -0.0501743046,
              0.0889789874, -0.2145988016, 1.5707963050):
        p = p * ax + jnp.float32(c)
    r = jnp.sqrt(1.0 - ax) * p
    return jnp.where(z < 0, jnp.float32(3.14159265358979) - r, r)


def _edge_angle_body(xi_ref, xj_ref, out_ref, *, be, n_edges):
    d = jnp.sum(xi_ref[...] * xj_ref[...], axis=1, keepdims=True)
    th = _acos(jnp.clip(d, -0.99, 0.99))
    row = pl.program_id(0) * be + lax.broadcasted_iota(jnp.int32, (be, 1), 0)
    th = jnp.where(row < n_edges, th, 0.0)
    out_ref[...] = jnp.broadcast_to(th, out_ref.shape)


def _rotate_body(f_ref, pa_ref, out_ref):
    f = f_ref[...]
    agg = pa_ref[0, :, 0:1] + pa_ref[1, :, 0:1]
    c = jnp.cos(agg)
    s = jnp.sin(agg)
    f0 = f[:, 0:1]
    f1 = f[:, 1:2]
    lane = lax.broadcasted_iota(jnp.int32, f.shape, 1)
    out_ref[...] = jnp.where(lane == 0, c * f0 - s * f1,
                             jnp.where(lane == 1, s * f0 + c * f1, f))


def _sc_gather4_body(feat_hbm, x_hbm, e0_hbm, e1_hbm,
                     fi_hbm, fj_hbm, ci_hbm, cj_hbm,
                     i0_v, i1_v, b0, b1, b2, b3, sem, *, nch):
    wid = lax.axis_index("s") * NC + lax.axis_index("c")
    nk = (nch - wid + (NW - 1)) // NW

    def body(t, carry):
        base = pl.multiple_of((wid + t * NW) * CH, CH)
        pltpu.sync_copy(e0_hbm.at[pl.ds(base, CH)], i0_v)
        pltpu.sync_copy(e1_hbm.at[pl.ds(base, CH)], i1_v)
        d0 = pltpu.async_copy(feat_hbm.at[i0_v], b0, sem)
        d1 = pltpu.async_copy(feat_hbm.at[i1_v], b1, sem)
        d2 = pltpu.async_copy(x_hbm.at[i0_v], b2, sem)
        d3 = pltpu.async_copy(x_hbm.at[i1_v], b3, sem)
        d0.wait()
        d1.wait()
        d2.wait()
        d3.wait()
        pltpu.sync_copy(b0, fi_hbm.at[pl.ds(base, CH)])
        pltpu.sync_copy(b1, fj_hbm.at[pl.ds(base, CH)])
        pltpu.sync_copy(b2, ci_hbm.at[pl.ds(base, CH)])
        pltpu.sync_copy(b3, cj_hbm.at[pl.ds(base, CH)])
        return carry

    lax.fori_loop(0, nk, body, 0)


def _sc_gather2_body(tab_hbm, e0_hbm, e1_hbm, xi_hbm, xj_hbm,
                     i0_v, i1_v, b0, b1, sem, *, nch):
    wid = lax.axis_index("s") * NC + lax.axis_index("c")
    nk = (nch - wid + (NW - 1)) // NW

    def body(t, carry):
        base = pl.multiple_of((wid + t * NW) * CH, CH)
        pltpu.sync_copy(e0_hbm.at[pl.ds(base, CH)], i0_v)
        pltpu.sync_copy(e1_hbm.at[pl.ds(base, CH)], i1_v)
        d0 = pltpu.async_copy(tab_hbm.at[i0_v], b0, sem)
        d1 = pltpu.async_copy(tab_hbm.at[i1_v], b1, sem)
        d0.wait()
        d1.wait()
        pltpu.sync_copy(b0, xi_hbm.at[pl.ds(base, CH)])
        pltpu.sync_copy(b1, xj_hbm.at[pl.ds(base, CH)])
        return carry

    lax.fori_loop(0, nk, body, 0)


def _sc_scatter_body(val_hbm, e0_hbm, zero_hbm, out_hbm,
                     idx_v, rows_v, accum, *, nch, rpt, width):
    cid = lax.axis_index("c")
    sid = lax.axis_index("s")
    wid = sid * NC + cid
    pltpu.sync_copy(zero_hbm, accum.at[pl.ds(sid * rpt, rpt)])
    plsc.subcore_barrier()
    nk = (nch - wid + (NW - 1)) // NW

    def body(t, carry):
        base = pl.multiple_of((wid + t * NW) * CH, CH)
        pltpu.sync_copy(e0_hbm.at[pl.ds(base, CH)], idx_v)
        pltpu.sync_copy(val_hbm.at[pl.ds(base, CH)], rows_v)
        pltpu.sync_copy(rows_v, accum.at[idx_v], add=True)
        return carry

    lax.fori_loop(0, nk, body, 0)
    plsc.subcore_barrier()
    pltpu.sync_copy(accum.at[pl.ds(sid * rpt, rpt)],
                    out_hbm.at[cid, pl.ds(sid * rpt, rpt)])


def _pad_rows(a, rows):
    if a.shape[0] == rows:
        return a
    pad = [(0, rows - a.shape[0])] + [(0, 0)] * (a.ndim - 1)
    return jnp.pad(a, pad)


def kernel(input, x, edge_index, W_lin, b_lin, gamma, beta,
           W1, b1, W2, b2, W3, b3, Wm, bm):
    n, d_in = input.shape
    hid = W_lin.shape[1]
    e = edge_index.shape[1]

    blkn = 512
    be = 1280
    np_ = ((n + 2559) // 2560) * 2560
    ep = ((e + be - 1) // be) * be
    nch = ep // CH
    rpt = np_ // NS
    ngrid = np_ // blkn
    egrid = ep // be

    e0 = _pad_rows(edge_index[0].astype(jnp.int32), ep)
    e1 = _pad_rows(edge_index[1].astype(jnp.int32), ep)
    inp_p = _pad_rows(input, np_)
    x_p = _pad_rows(x, np_)

    bl = b_lin.reshape(1, hid)
    ga = gamma.reshape(1, hid)
    bt = beta.reshape(1, hid)
    w1a = W1[:hid]
    w1b = W1[hid:]
    w2r0 = W2[0].reshape(1, hid)
    w2r1 = W2[1].reshape(1, hid)
    w3r0 = W3[0].reshape(1, hid)
    w3r1 = W3[1].reshape(1, hid)
    bsum = (b1 + b2 + b3).reshape(1, hid)
    bmr = bm.reshape(1, hid)

    rep = lambda shape: pl.BlockSpec(shape, lambda i: (0,) * len(shape))

    feat = pl.pallas_call(
        _node_feat_body,
        grid=(ngrid,),
        in_specs=[pl.BlockSpec((blkn, d_in), lambda i: (i, 0)),
                  rep((d_in, hid)), rep((1, hid)), rep((1, hid)),
                  rep((1, hid))],
        out_specs=pl.BlockSpec((blkn, hid), lambda i: (i, 0)),
        out_shape=jax.ShapeDtypeStruct((np_, hid), F32),
    )(inp_p, W_lin, bl, ga, bt)

    mesh = plsc.VectorSubcoreMesh(core_axis_name="c", subcore_axis_name="s",
                                  num_cores=NC, num_subcores=NS)

    gather4 = pl.kernel(
        functools.partial(_sc_gather4_body, nch=nch),
        out_type=[jax.ShapeDtypeStruct((ep, hid), F32)] * 4,
        mesh=mesh,
        scratch_types=[pltpu.VMEM((CH,), jnp.int32),
                       pltpu.VMEM((CH,), jnp.int32),
                       pltpu.VMEM((CH, hid), F32),
                       pltpu.VMEM((CH, hid), F32),
                       pltpu.VMEM((CH, hid), F32),
                       pltpu.VMEM((CH, hid), F32),
                       pltpu.SemaphoreType.DMA],
    )
    fi, fj, ci, cj = gather4(feat, x_p, e0, e1)

    update = pl.pallas_call(
        _edge_update_body,
        grid=(egrid,),
        in_specs=[pl.BlockSpec((be, hid), lambda i: (i, 0))] * 4 +
                 [rep((hid, hid)), rep((hid, hid)), rep((hid, hid)),
                  rep((1, hid)), rep((1, hid)), rep((1, hid)), rep((1, hid)),
                  rep((1, hid)), rep((1, hid))],
        out_specs=pl.BlockSpec((be, hid), lambda i: (i, 0)),
        out_shape=jax.ShapeDtypeStruct((ep, hid), F32),
    )(fi, fj, ci, cj, w1a, w1b, Wm, w2r0, w2r1, w3r0, w3r1, bsum, bmr)

    zeros_w = jnp.zeros((rpt, hid), F32)
    scatter_u = pl.kernel(
        functools.partial(_sc_scatter_body, nch=nch, rpt=rpt, width=hid),
        out_type=jax.ShapeDtypeStruct((NC, np_, hid), F32),
        mesh=mesh,
        scratch_types=[pltpu.VMEM((CH,), jnp.int32),
                       pltpu.VMEM((CH, hid), F32),
                       pltpu.VMEM_SHARED((np_, hid), F32)],
    )
    partials = scatter_u(update, e0, zeros_w)

    xn2 = pl.pallas_call(
        _normalize_body,
        grid=(ngrid,),
        in_specs=[pl.BlockSpec((blkn, hid), lambda i: (i, 0)),
                  pl.BlockSpec((NC, blkn, hid), lambda i: (0, i, 0))],
        out_specs=pl.BlockSpec((blkn, hid), lambda i: (i, 0)),
        out_shape=jax.ShapeDtypeStruct((np_, hid), F32),
    )(x_p, partials)

    gather2 = pl.kernel(
        functools.partial(_sc_gather2_body, nch=nch),
        out_type=[jax.ShapeDtypeStruct((ep, hid), F32)] * 2,
        mesh=mesh,
        scratch_types=[pltpu.VMEM((CH,), jnp.int32),
                       pltpu.VMEM((CH,), jnp.int32),
                       pltpu.VMEM((CH, hid), F32),
                       pltpu.VMEM((CH, hid), F32),
                       pltpu.SemaphoreType.DMA],
    )
    xi2, xj2 = gather2(xn2, e0, e1)

    vals = pl.pallas_call(
        functools.partial(_edge_angle_body, be=be, n_edges=e),
        grid=(egrid,),
        in_specs=[pl.BlockSpec((be, hid), lambda i: (i, 0))] * 2,
        out_specs=pl.BlockSpec((be, hid), lambda i: (i, 0)),
        out_shape=jax.ShapeDtypeStruct((ep, hid), F32),
    )(xi2, xj2)

    scatter_a = pl.kernel(
        functools.partial(_sc_scatter_body, nch=nch, rpt=rpt, width=hid),
        out_type=jax.ShapeDtypeStruct((NC, np_, hid), F32),
        mesh=mesh,
        scratch_types=[pltpu.VMEM((CH,), jnp.int32),
                       pltpu.VMEM((CH, hid), F32),
                       pltpu.VMEM_SHARED((np_, hid), F32)],
    )
    agg_p = scatter_a(vals, e0, zeros_w)

    out = pl.pallas_call(
        _rotate_body,
        grid=(ngrid,),
        in_specs=[pl.BlockSpec((blkn, hid), lambda i: (i, 0)),
                  pl.BlockSpec((NC, blkn, hid), lambda i: (0, i, 0))],
        out_specs=pl.BlockSpec((blkn, hid), lambda i: (i, 0)),
        out_shape=jax.ShapeDtypeStruct((np_, hid), F32),
    )(feat, agg_p)

    return out[:n]
